# Initial kernel scaffold; baseline (speedup 1.0000x reference)
import functools

import jax
import jax.numpy as jnp
from jax import lax
from jax.experimental import pallas as pl
from jax.experimental.pallas import tpu as pltpu
from jax.experimental.pallas import tpu_sc as plsc

N = 10000
DEG = 16
F_IN = 128
HID = 64
N_CLS = 10

NC = 2
NS = 16
NW = NC * NS
NP = 10240
PER_W = NP // NW
LANES = 16
LUT = 320

_mesh = plsc.VectorSubcoreMesh(
    core_axis_name="c", subcore_axis_name="s", num_cores=NC, num_subcores=NS)


def _iota():
    return lax.iota(jnp.int32, LANES)


def _full(v):
    return jnp.full((LANES,), v, jnp.int32)


@functools.partial(
    pl.kernel,
    out_type=(
        jax.ShapeDtypeStruct((NP, LANES, LANES), jnp.int32),
        jax.ShapeDtypeStruct((NP * DEG,), jnp.float32),
        jax.ShapeDtypeStruct((NP * 256,), jnp.float32),
        jax.ShapeDtypeStruct((NP,), jnp.float32),
        jax.ShapeDtypeStruct((NP,), jnp.float32),
    ),
    mesh=_mesh,
    scratch_types=(
        pltpu.VMEM((NP,), jnp.int32),
        pltpu.VMEM((NP,), jnp.int32),
        pltpu.VMEM((PER_W * DEG,), jnp.int32),
        pltpu.VMEM((LANES, LANES), jnp.int32),
        pltpu.VMEM((256,), jnp.float32),
        pltpu.VMEM((PER_W * DEG,), jnp.float32),
        pltpu.VMEM((PER_W,), jnp.float32),
        pltpu.VMEM((PER_W,), jnp.float32),
        pltpu.VMEM((LUT,), jnp.float32),
        pltpu.VMEM((LANES,), jnp.int32),
        pltpu.SemaphoreType.DMA,
    ),
)
def _structure_kernel(nbr2d, nbrflat, lut_hbm, cand_out, w1_out, w2_out,
                      d1_out, d2_out, cnt, slot, nbrslab, cand2d, w2buf,
                      w1slab, d1slab, d2slab, lutv, idx16, sem):
    wid = lax.axis_index("s") * NC + lax.axis_index("c")
    base = wid * PER_W
    pltpu.sync_copy(lut_hbm, lutv)
    pltpu.sync_copy(nbrflat.at[pl.ds(base * DEG, PER_W * DEG)], nbrslab)

    iota = _iota()
    lane0 = iota == 0
    zeros_i = jnp.zeros((LANES,), jnp.int32)
    ones_i = jnp.ones((LANES,), jnp.int32)

    def _zb(j, _):
        cnt[pl.ds(j * LANES, LANES)] = zeros_i
        return 0
    lax.fori_loop(0, NP // LANES, _zb, 0)

    def body(li, _):
        i = base + li
        i_spl = _full(i)
        v = nbrslab[pl.ds(li * DEG, DEG)]
        idx16[...] = v
        cp = pltpu.async_copy(nbr2d.at[idx16], cand2d, sem)

        plsc.addupdate_scatter(cnt, [v], ones_i)
        g = plsc.load_gather(cnt, [v])
        plsc.store_scatter(slot, [v], iota)
        back = plsc.load_gather(slot, [v])
        chosen = back == iota
        g_adj = g - jnp.where(v == i_spl, 1, 0)
        valid1 = chosen & (g_adj > 0)
        w1v = jnp.where(valid1, 1.0, 0.0)
        w1slab[pl.ds(li * DEG, DEG)] = w1v
        deg1 = plsc.all_reduce_population_count(valid1)
        plsc.store_scatter(cnt, [v], zeros_i)

        cp.wait()

        for s in range(16):
            cv = plsc.load_gather(cand2d, [_full(s), iota])
            plsc.addupdate_scatter(cnt, [cv], ones_i)
        plsc.addupdate_scatter(cnt, [v], -ones_i)
        plsc.addupdate_scatter(cnt, [i_spl], -ones_i, mask=lane0)
        for s in range(16):
            cv = plsc.load_gather(cand2d, [_full(s), iota])
            plsc.store_scatter(slot, [cv], iota + 16 * s)
        deg2 = jnp.zeros((LANES,), jnp.int32)
        for s in range(16):
            cv = plsc.load_gather(cand2d, [_full(s), iota])
            g2 = plsc.load_gather(cnt, [cv])
            b2 = plsc.load_gather(slot, [cv])
            m = (b2 == iota + 16 * s) & (g2 > 0)
            w2buf[pl.ds(s * LANES, LANES)] = jnp.where(m, 1.0, 0.0)
            deg2 = deg2 + plsc.all_reduce_population_count(m)
        for s in range(16):
            cv = plsc.load_gather(cand2d, [_full(s), iota])
            plsc.store_scatter(cnt, [cv], zeros_i)
        plsc.store_scatter(cnt, [v], zeros_i)
        plsc.store_scatter(cnt, [i_spl], zeros_i, mask=lane0)

        d1s = plsc.load_gather(lutv, [deg1])
        d2s = plsc.load_gather(lutv, [deg2])
        plsc.store_scatter(d1slab, [_full(li)], d1s, mask=lane0)
        plsc.store_scatter(d2slab, [_full(li)], d2s, mask=lane0)

        pltpu.sync_copy(cand2d, cand_out.at[i])
        pltpu.sync_copy(w2buf, w2_out.at[pl.ds(i * 256, 256)])
        return 0

    lax.fori_loop(0, PER_W, body, 0)
    pltpu.sync_copy(w1slab, w1_out.at[pl.ds(base * DEG, PER_W * DEG)])
    pltpu.sync_copy(d1slab, d1_out.at[pl.ds(base, PER_W)])
    pltpu.sync_copy(d2slab, d2_out.at[pl.ds(base, PER_W)])


def _make_layer_kernel(W):
    WL = W // LANES

    @functools.partial(
        pl.kernel,
        out_type=jax.ShapeDtypeStruct((NP * 2 * W,), jnp.float32),
        mesh=_mesh,
        scratch_types=(
            pltpu.VMEM((NP,), jnp.float32),
            pltpu.VMEM((NP,), jnp.float32),
            pltpu.VMEM((PER_W * DEG,), jnp.int32),
            pltpu.VMEM((PER_W * DEG,), jnp.float32),
            pltpu.VMEM((256,), jnp.int32),
            pltpu.VMEM((256,), jnp.float32),
            pltpu.VMEM((256,), jnp.float32),
            pltpu.VMEM((LANES,), jnp.float32),
            pltpu.VMEM((LANES,), jnp.int32),
            pltpu.VMEM((DEG, W), jnp.float32),
            pltpu.VMEM((256, W), jnp.float32),
            pltpu.VMEM((2 * W,), jnp.float32),
            pltpu.SemaphoreType.DMA,
            pltpu.SemaphoreType.DMA,
        ),
    )
    def layer(table, nbrflat, candflat, w1flat, w2flat, d1_hbm, d2_hbm,
              out_hbm, d1v, d2v, nbrslab, w1slab, idx256, w2buf, coef2,
              coef1, idx16, rows16, rows256, outbuf, semA, semB):
        wid = lax.axis_index("s") * NC + lax.axis_index("c")
        base = wid * PER_W
        pltpu.sync_copy(d1_hbm, d1v)
        pltpu.sync_copy(d2_hbm, d2v)
        pltpu.sync_copy(nbrflat.at[pl.ds(base * DEG, PER_W * DEG)], nbrslab)
        pltpu.sync_copy(w1flat.at[pl.ds(base * DEG, PER_W * DEG)], w1slab)

        iota = _iota()

        def body(li, _):
            i = base + li
            pltpu.sync_copy(candflat.at[pl.ds(i * 256, 256)], idx256)
            pltpu.sync_copy(w2flat.at[pl.ds(i * 256, 256)], w2buf)
            v = nbrslab[pl.ds(li * DEG, DEG)]
            idx16[...] = v
            cpA = pltpu.async_copy(table.at[idx16], rows16, semA)
            cpB = pltpu.async_copy(table.at[idx256.at[pl.ds(0, 128)]],
                                   rows256.at[pl.ds(0, 128)], semB)
            cpC = pltpu.async_copy(table.at[idx256.at[pl.ds(128, 128)]],
                                   rows256.at[pl.ds(128, 128)], semB)
            coef1[...] = w1slab[pl.ds(li * DEG, DEG)] * plsc.load_gather(d1v, [v])

            def cg(g, _):
                cv = idx256[pl.ds(g * LANES, LANES)]
                coef2[pl.ds(g * LANES, LANES)] = (
                    w2buf[pl.ds(g * LANES, LANES)] * plsc.load_gather(d2v, [cv]))
                return 0
            lax.fori_loop(0, 16, cg, 0)

            cpA.wait()
            acc1 = [jnp.zeros((LANES,), jnp.float32) for _ in range(WL)]
            for t in range(DEG):
                sp = plsc.load_gather(coef1, [_full(t)])
                for l in range(WL):
                    acc1[l] = acc1[l] + sp * plsc.load_gather(
                        rows16, [_full(t), iota + l * LANES])

            cpB.wait()
            cpC.wait()

            def ag(g, accs):
                accs = list(accs)
                for t in range(16):
                    slot = g * 16 + t
                    sp = plsc.load_gather(coef2, [_full(slot)])
                    for l in range(WL):
                        accs[l] = accs[l] + sp * plsc.load_gather(
                            rows256, [_full(slot), iota + l * LANES])
                return tuple(accs)
            acc2 = lax.fori_loop(
                0, 16, ag,
                tuple(jnp.zeros((LANES,), jnp.float32) for _ in range(WL)))

            d1i = plsc.load_gather(d1v, [_full(i)])
            d2i = plsc.load_gather(d2v, [_full(i)])
            for l in range(WL):
                outbuf[pl.ds(l * LANES, LANES)] = jnp.maximum(acc1[l] * d1i, 0.0)
                outbuf[pl.ds(W + l * LANES, LANES)] = jnp.maximum(
                    acc2[l] * d2i, 0.0)
            pltpu.sync_copy(outbuf, out_hbm.at[pl.ds(i * 2 * W, 2 * W)])
            return 0

        lax.fori_loop(0, PER_W, body, 0)

    return layer


_layer64 = _make_layer_kernel(HID)
_layer128 = _make_layer_kernel(2 * HID)


_BK = 1024


def _embed_body(x_ref, w_ref, o_ref):
    o_ref[...] = jnp.maximum(
        jnp.dot(x_ref[...], w_ref[...], preferred_element_type=jnp.float32),
        0.0)


def _classify_body(r0_ref, r1_ref, r2_ref, w_ref, o_ref):
    w = w_ref[...]
    lg = jnp.dot(r0_ref[...], w[0:HID],
                 preferred_element_type=jnp.float32)
    lg = lg + jnp.dot(r1_ref[...], w[HID:3 * HID],
                      preferred_element_type=jnp.float32)
    lg = lg + jnp.dot(r2_ref[...], w[3 * HID:7 * HID],
                      preferred_element_type=jnp.float32)
    m = jnp.max(lg, axis=1, keepdims=True)
    e = jnp.exp(lg - m)
    o_ref[...] = e / jnp.sum(e, axis=1, keepdims=True)


def kernel(x, edge_index, w_embed, w_classify):
    src = edge_index[1].astype(jnp.int32)
    nbr2d = jnp.zeros((NP, DEG), jnp.int32).at[:N].set(src.reshape(N, DEG))
    nbrflat = nbr2d.reshape(-1)
    lut = jnp.where(jnp.arange(LUT) > 0,
                    jnp.arange(LUT, dtype=jnp.float32) ** -0.5,
                    0.0).astype(jnp.float32)
    x_pad = jnp.zeros((NP, F_IN), x.dtype).at[:N].set(x)

    r0 = pl.pallas_call(
        _embed_body,
        grid=(NP // _BK,),
        in_specs=[
            pl.BlockSpec((_BK, F_IN), lambda g: (g, 0)),
            pl.BlockSpec((F_IN, HID), lambda g: (0, 0)),
        ],
        out_specs=pl.BlockSpec((_BK, HID), lambda g: (g, 0)),
        out_shape=jax.ShapeDtypeStruct((NP, HID), jnp.float32),
    )(x_pad, w_embed)

    cand3d, w1f, w2f, d1, d2 = _structure_kernel(nbr2d, nbrflat, lut)
    candflat = cand3d.reshape(-1)

    r1 = _layer64(r0, nbrflat, candflat, w1f, w2f, d1, d2).reshape(NP, 2 * HID)
    r2 = _layer128(r1, nbrflat, candflat, w1f, w2f, d1, d2).reshape(NP, 4 * HID)

    out = pl.pallas_call(
        _classify_body,
        grid=(NP // _BK,),
        in_specs=[
            pl.BlockSpec((_BK, HID), lambda g: (g, 0)),
            pl.BlockSpec((_BK, 2 * HID), lambda g: (g, 0)),
            pl.BlockSpec((_BK, 4 * HID), lambda g: (g, 0)),
            pl.BlockSpec((7 * HID, N_CLS), lambda g: (0, 0)),
        ],
        out_specs=pl.BlockSpec((_BK, N_CLS), lambda g: (g, 0)),
        out_shape=jax.ShapeDtypeStruct((NP, N_CLS), jnp.float32),
    )(r0, r1, r2, w_classify)

    return out[:N]


# speedup vs baseline: 28.1878x; 28.1878x over previous
import functools

import jax
import jax.numpy as jnp
from jax import lax
from jax.experimental import pallas as pl
from jax.experimental.pallas import tpu as pltpu
from jax.experimental.pallas import tpu_sc as plsc

N = 10000
DEG = 16
F_IN = 128
HID = 64
N_CLS = 10

NC = 2
NS = 16
NW = NC * NS
NP = 10240
PER_W = NP // NW
LANES = 16
LUT = 320

_mesh = plsc.VectorSubcoreMesh(
    core_axis_name="c", subcore_axis_name="s", num_cores=NC, num_subcores=NS)
_sc_params = pltpu.CompilerParams(
    needs_layout_passes=False, use_tc_tiling_on_sc=False)


def _iota():
    return lax.iota(jnp.int32, LANES)


def _full(v):
    return jnp.full((LANES,), v, jnp.int32)


@functools.partial(
    pl.kernel,
    out_type=(
        jax.ShapeDtypeStruct((NP, 256), jnp.int32),
        jax.ShapeDtypeStruct((NP, DEG), jnp.float32),
        jax.ShapeDtypeStruct((NP, 256), jnp.float32),
        jax.ShapeDtypeStruct((NP,), jnp.float32),
        jax.ShapeDtypeStruct((NP,), jnp.float32),
    ),
    mesh=_mesh,
    compiler_params=_sc_params,
    scratch_types=(
        pltpu.VMEM((NP,), jnp.int32),
        pltpu.VMEM((NP,), jnp.int32),
        pltpu.VMEM((PER_W, DEG), jnp.int32),
        pltpu.VMEM((LANES, LANES), jnp.int32),
        pltpu.VMEM((256,), jnp.int32),
        pltpu.VMEM((256,), jnp.float32),
        pltpu.VMEM((PER_W, DEG), jnp.float32),
        pltpu.VMEM((PER_W,), jnp.float32),
        pltpu.VMEM((PER_W,), jnp.float32),
        pltpu.VMEM((LUT,), jnp.float32),
        pltpu.VMEM((LANES,), jnp.int32),
        pltpu.SemaphoreType.DMA,
    ),
)
def _structure_kernel(nbr2d, lut_hbm, cand_out, w1_out, w2_out,
                      d1_out, d2_out, cnt, slot, nbrslab, cand2d, candbuf,
                      w2buf, w1slab, d1slab, d2slab, lutv, idx16, sem):
    wid = lax.axis_index("s") * NC + lax.axis_index("c")
    base = wid * PER_W
    pltpu.sync_copy(lut_hbm, lutv)
    pltpu.sync_copy(nbr2d.at[pl.ds(base, PER_W)], nbrslab)

    iota = _iota()
    lane0 = iota == 0
    zeros_i = jnp.zeros((LANES,), jnp.int32)
    ones_i = jnp.ones((LANES,), jnp.int32)

    def _zb(j, _):
        plsc.store_scatter(cnt, [iota + j * LANES], zeros_i)
        return 0
    lax.fori_loop(0, NP // LANES, _zb, 0)

    def body(li, _):
        i = base + li
        i_spl = _full(i)
        v = plsc.load_gather(nbrslab, [_full(li), iota])
        idx16[...] = v
        cp = pltpu.async_copy(nbr2d.at[idx16], cand2d, sem)

        plsc.addupdate_scatter(cnt, [v], ones_i)
        g = plsc.load_gather(cnt, [v])
        plsc.store_scatter(slot, [v], iota)
        back = plsc.load_gather(slot, [v])
        chosen = back == iota
        g_adj = g - jnp.where(v == i_spl, 1, 0)
        valid1 = chosen & (g_adj > 0)
        w1v = jnp.where(valid1, 1.0, 0.0)
        plsc.store_scatter(w1slab, [_full(li), iota], w1v)
        deg1 = plsc.all_reduce_population_count(valid1)
        plsc.store_scatter(cnt, [v], zeros_i)

        cp.wait()

        for s in range(16):
            cv = plsc.load_gather(cand2d, [_full(s), iota])
            candbuf[pl.ds(s * LANES, LANES)] = cv
            plsc.addupdate_scatter(cnt, [cv], ones_i)
        plsc.addupdate_scatter(cnt, [v], -ones_i)
        plsc.addupdate_scatter(cnt, [i_spl], -ones_i, mask=lane0)
        for s in range(16):
            cv = candbuf[pl.ds(s * LANES, LANES)]
            plsc.store_scatter(slot, [cv], iota + 16 * s)
        deg2 = jnp.zeros((LANES,), jnp.int32)
        for s in range(16):
            cv = candbuf[pl.ds(s * LANES, LANES)]
            g2 = plsc.load_gather(cnt, [cv])
            b2 = plsc.load_gather(slot, [cv])
            m = (b2 == iota + 16 * s) & (g2 > 0)
            w2buf[pl.ds(s * LANES, LANES)] = jnp.where(m, 1.0, 0.0)
            deg2 = deg2 + plsc.all_reduce_population_count(m)
        for s in range(16):
            cv = candbuf[pl.ds(s * LANES, LANES)]
            plsc.store_scatter(cnt, [cv], zeros_i)
        plsc.store_scatter(cnt, [v], zeros_i)
        plsc.store_scatter(cnt, [i_spl], zeros_i, mask=lane0)

        d1s = plsc.load_gather(lutv, [deg1])
        d2s = plsc.load_gather(lutv, [deg2])
        plsc.store_scatter(d1slab, [_full(li)], d1s, mask=lane0)
        plsc.store_scatter(d2slab, [_full(li)], d2s, mask=lane0)

        pltpu.sync_copy(candbuf, cand_out.at[i])
        pltpu.sync_copy(w2buf, w2_out.at[i])
        return 0

    lax.fori_loop(0, PER_W, body, 0)
    pltpu.sync_copy(w1slab, w1_out.at[pl.ds(base, PER_W)])
    pltpu.sync_copy(d1slab, d1_out.at[pl.ds(base, PER_W)])
    pltpu.sync_copy(d2slab, d2_out.at[pl.ds(base, PER_W)])


def _make_layer_kernel(W):
    WL = W // LANES

    @functools.partial(
        pl.kernel,
        out_type=jax.ShapeDtypeStruct((NP, 2 * W), jnp.float32),
        mesh=_mesh,
        compiler_params=_sc_params,
        scratch_types=(
            pltpu.VMEM((NP,), jnp.float32),
            pltpu.VMEM((NP,), jnp.float32),
            pltpu.VMEM((PER_W, DEG), jnp.int32),
            pltpu.VMEM((PER_W, DEG), jnp.float32),
            pltpu.VMEM((256,), jnp.int32),
            pltpu.VMEM((256,), jnp.float32),
            pltpu.VMEM((256,), jnp.float32),
            pltpu.VMEM((LANES,), jnp.float32),
            pltpu.VMEM((LANES,), jnp.int32),
            pltpu.VMEM((DEG, W), jnp.float32),
            pltpu.VMEM((256, W), jnp.float32),
            pltpu.VMEM((2 * W,), jnp.float32),
            pltpu.SemaphoreType.DMA,
            pltpu.SemaphoreType.DMA,
        ),
    )
    def layer(table, nbr2d, cand, w1_in, w2_in, d1_hbm, d2_hbm,
              out_hbm, d1v, d2v, nbrslab, w1slab, idx256, w2buf, coef2,
              coef1, idx16, rows16, rows256, outbuf, semA, semB):
        wid = lax.axis_index("s") * NC + lax.axis_index("c")
        base = wid * PER_W
        pltpu.sync_copy(d1_hbm, d1v)
        pltpu.sync_copy(d2_hbm, d2v)
        pltpu.sync_copy(nbr2d.at[pl.ds(base, PER_W)], nbrslab)
        pltpu.sync_copy(w1_in.at[pl.ds(base, PER_W)], w1slab)

        iota = _iota()

        def body(li, _):
            i = base + li
            pltpu.sync_copy(cand.at[i], idx256)
            pltpu.sync_copy(w2_in.at[i], w2buf)
            v = plsc.load_gather(nbrslab, [_full(li), iota])
            idx16[...] = v
            cpA = pltpu.async_copy(table.at[idx16], rows16, semA)
            cpB = pltpu.async_copy(table.at[idx256.at[pl.ds(0, 128)]],
                                   rows256.at[pl.ds(0, 128)], semB)
            cpC = pltpu.async_copy(table.at[idx256.at[pl.ds(128, 128)]],
                                   rows256.at[pl.ds(128, 128)], semB)
            w1v = plsc.load_gather(w1slab, [_full(li), iota])
            coef1[...] = w1v * plsc.load_gather(d1v, [v])

            def cg(g, _):
                cv = idx256[pl.ds(g * LANES, LANES)]
                coef2[pl.ds(g * LANES, LANES)] = (
                    w2buf[pl.ds(g * LANES, LANES)] * plsc.load_gather(d2v, [cv]))
                return 0
            lax.fori_loop(0, 16, cg, 0)

            cpA.wait()
            acc1 = [jnp.zeros((LANES,), jnp.float32) for _ in range(WL)]
            for t in range(DEG):
                sp = plsc.load_gather(coef1, [_full(t)])
                for l in range(WL):
                    acc1[l] = acc1[l] + sp * plsc.load_gather(
                        rows16, [_full(t), iota + l * LANES])

            cpB.wait()
            cpC.wait()

            def ag(g, accs):
                accs = list(accs)
                for t in range(16):
                    slot = g * 16 + t
                    sp = plsc.load_gather(coef2, [_full(slot)])
                    for l in range(WL):
                        accs[l] = accs[l] + sp * plsc.load_gather(
                            rows256, [_full(slot), iota + l * LANES])
                return tuple(accs)
            acc2 = lax.fori_loop(
                0, 16, ag,
                tuple(jnp.zeros((LANES,), jnp.float32) for _ in range(WL)))

            d1i = plsc.load_gather(d1v, [_full(i)])
            d2i = plsc.load_gather(d2v, [_full(i)])
            for l in range(WL):
                outbuf[pl.ds(l * LANES, LANES)] = jnp.maximum(acc1[l] * d1i, 0.0)
                outbuf[pl.ds(W + l * LANES, LANES)] = jnp.maximum(
                    acc2[l] * d2i, 0.0)
            pltpu.sync_copy(outbuf, out_hbm.at[i])
            return 0

        lax.fori_loop(0, PER_W, body, 0)

    return layer


_layer64 = _make_layer_kernel(HID)
_layer128 = _make_layer_kernel(2 * HID)


_BK = 1024


def _embed_body(x_ref, w_ref, o_ref):
    o_ref[...] = jnp.maximum(
        jnp.dot(x_ref[...], w_ref[...], preferred_element_type=jnp.float32),
        0.0)


def _classify_body(r0_ref, r1_ref, r2_ref, w_ref, o_ref):
    w = w_ref[...]
    lg = jnp.dot(r0_ref[...], w[0:HID],
                 preferred_element_type=jnp.float32)
    lg = lg + jnp.dot(r1_ref[...], w[HID:3 * HID],
                      preferred_element_type=jnp.float32)
    lg = lg + jnp.dot(r2_ref[...], w[3 * HID:7 * HID],
                      preferred_element_type=jnp.float32)
    m = jnp.max(lg, axis=1, keepdims=True)
    e = jnp.exp(lg - m)
    o_ref[...] = e / jnp.sum(e, axis=1, keepdims=True)


def kernel(x, edge_index, w_embed, w_classify):
    src = edge_index[1].astype(jnp.int32)
    nbr2d = jnp.zeros((NP, DEG), jnp.int32).at[:N].set(src.reshape(N, DEG))
    lut = jnp.where(jnp.arange(LUT) > 0,
                    jnp.arange(LUT, dtype=jnp.float32) ** -0.5,
                    0.0).astype(jnp.float32)
    x_pad = jnp.zeros((NP, F_IN), x.dtype).at[:N].set(x)

    r0 = pl.pallas_call(
        _embed_body,
        grid=(NP // _BK,),
        in_specs=[
            pl.BlockSpec((_BK, F_IN), lambda g: (g, 0)),
            pl.BlockSpec((F_IN, HID), lambda g: (0, 0)),
        ],
        out_specs=pl.BlockSpec((_BK, HID), lambda g: (g, 0)),
        out_shape=jax.ShapeDtypeStruct((NP, HID), jnp.float32),
    )(x_pad, w_embed)

    cand, w1, w2, d1, d2 = _structure_kernel(nbr2d, lut)

    r1 = _layer64(r0, nbr2d, cand, w1, w2, d1, d2)
    r2 = _layer128(r1, nbr2d, cand, w1, w2, d1, d2)

    out = pl.pallas_call(
        _classify_body,
        grid=(NP // _BK,),
        in_specs=[
            pl.BlockSpec((_BK, HID), lambda g: (g, 0)),
            pl.BlockSpec((_BK, 2 * HID), lambda g: (g, 0)),
            pl.BlockSpec((_BK, 4 * HID), lambda g: (g, 0)),
            pl.BlockSpec((7 * HID, N_CLS), lambda g: (0, 0)),
        ],
        out_specs=pl.BlockSpec((_BK, N_CLS), lambda g: (g, 0)),
        out_shape=jax.ShapeDtypeStruct((NP, N_CLS), jnp.float32),
    )(r0, r1, r2, w_classify)

    return out[:N]


# speedup vs baseline: 46.9494x; 1.6656x over previous
import functools

import jax
import jax.numpy as jnp
from jax import lax
from jax.experimental import pallas as pl
from jax.experimental.pallas import tpu as pltpu
from jax.experimental.pallas import tpu_sc as plsc

N = 10000
DEG = 16
F_IN = 128
HID = 64
N_CLS = 10

NC = 2
NS = 16
NW = NC * NS
NP = 10240
PER_W = NP // NW
LANES = 16
LUT = 320

_mesh = plsc.VectorSubcoreMesh(
    core_axis_name="c", subcore_axis_name="s", num_cores=NC, num_subcores=NS)
_sc_params = pltpu.CompilerParams(
    needs_layout_passes=False, use_tc_tiling_on_sc=False)


def _iota():
    return lax.iota(jnp.int32, LANES)


def _full(v):
    return jnp.full((LANES,), v, jnp.int32)


@functools.partial(
    pl.kernel,
    out_type=(
        jax.ShapeDtypeStruct((NP, 256), jnp.int32),
        jax.ShapeDtypeStruct((NP, DEG), jnp.float32),
        jax.ShapeDtypeStruct((NP, 256), jnp.float32),
        jax.ShapeDtypeStruct((NP,), jnp.float32),
        jax.ShapeDtypeStruct((NP,), jnp.float32),
    ),
    mesh=_mesh,
    compiler_params=_sc_params,
    scratch_types=(
        pltpu.VMEM((NP,), jnp.int32),
        pltpu.VMEM((NP,), jnp.int32),
        pltpu.VMEM((PER_W, DEG), jnp.int32),
        pltpu.VMEM((LANES, LANES), jnp.int32),
        pltpu.VMEM((256,), jnp.int32),
        pltpu.VMEM((256,), jnp.float32),
        pltpu.VMEM((PER_W, DEG), jnp.float32),
        pltpu.VMEM((PER_W,), jnp.float32),
        pltpu.VMEM((PER_W,), jnp.float32),
        pltpu.VMEM((LUT,), jnp.float32),
        pltpu.VMEM((LANES,), jnp.int32),
        pltpu.SemaphoreType.DMA,
    ),
)
def _structure_kernel(nbr2d, lut_hbm, cand_out, w1_out, w2_out,
                      d1_out, d2_out, cnt, slot, nbrslab, cand2d, candbuf,
                      w2buf, w1slab, d1slab, d2slab, lutv, idx16, sem):
    wid = lax.axis_index("s") * NC + lax.axis_index("c")
    base = wid * PER_W
    pltpu.sync_copy(lut_hbm, lutv)
    pltpu.sync_copy(nbr2d.at[pl.ds(base, PER_W)], nbrslab)

    iota = _iota()
    lane0 = iota == 0
    zeros_i = jnp.zeros((LANES,), jnp.int32)
    ones_i = jnp.ones((LANES,), jnp.int32)

    def _zb(j, _):
        plsc.store_scatter(cnt, [iota + j * LANES], zeros_i)
        return 0
    lax.fori_loop(0, NP // LANES, _zb, 0)

    def body(li, _):
        i = base + li
        i_spl = _full(i)
        v = plsc.load_gather(nbrslab, [_full(li), iota])
        idx16[...] = v
        cp = pltpu.async_copy(nbr2d.at[idx16], cand2d, sem)

        plsc.addupdate_scatter(cnt, [v], ones_i)
        g = plsc.load_gather(cnt, [v])
        plsc.store_scatter(slot, [v], iota)
        back = plsc.load_gather(slot, [v])
        chosen = back == iota
        g_adj = g - jnp.where(v == i_spl, 1, 0)
        valid1 = chosen & (g_adj > 0)
        w1v = jnp.where(valid1, 1.0, 0.0)
        plsc.store_scatter(w1slab, [_full(li), iota], w1v)
        deg1 = plsc.all_reduce_population_count(valid1)
        plsc.store_scatter(cnt, [v], zeros_i)

        cp.wait()

        for s in range(16):
            cv = plsc.load_gather(cand2d, [_full(s), iota])
            candbuf[pl.ds(s * LANES, LANES)] = cv
            plsc.addupdate_scatter(cnt, [cv], ones_i)
        plsc.addupdate_scatter(cnt, [v], -ones_i)
        plsc.addupdate_scatter(cnt, [i_spl], -ones_i, mask=lane0)
        for s in range(16):
            cv = candbuf[pl.ds(s * LANES, LANES)]
            plsc.store_scatter(slot, [cv], iota + 16 * s)
        deg2 = jnp.zeros((LANES,), jnp.int32)
        for s in range(16):
            cv = candbuf[pl.ds(s * LANES, LANES)]
            g2 = plsc.load_gather(cnt, [cv])
            b2 = plsc.load_gather(slot, [cv])
            m = (b2 == iota + 16 * s) & (g2 > 0)
            w2buf[pl.ds(s * LANES, LANES)] = jnp.where(m, 1.0, 0.0)
            deg2 = deg2 + plsc.all_reduce_population_count(m)
        for s in range(16):
            cv = candbuf[pl.ds(s * LANES, LANES)]
            plsc.store_scatter(cnt, [cv], zeros_i)
        plsc.store_scatter(cnt, [v], zeros_i)
        plsc.store_scatter(cnt, [i_spl], zeros_i, mask=lane0)

        d1s = plsc.load_gather(lutv, [deg1])
        d2s = plsc.load_gather(lutv, [deg2])
        plsc.store_scatter(d1slab, [_full(li)], d1s, mask=lane0)
        plsc.store_scatter(d2slab, [_full(li)], d2s, mask=lane0)

        pltpu.sync_copy(candbuf, cand_out.at[i])
        pltpu.sync_copy(w2buf, w2_out.at[i])
        return 0

    lax.fori_loop(0, PER_W, body, 0)
    pltpu.sync_copy(w1slab, w1_out.at[pl.ds(base, PER_W)])
    pltpu.sync_copy(d1slab, d1_out.at[pl.ds(base, PER_W)])
    pltpu.sync_copy(d2slab, d2_out.at[pl.ds(base, PER_W)])


def _make_layer_kernel(W):
    WL = W // LANES

    @functools.partial(
        pl.kernel,
        out_type=jax.ShapeDtypeStruct((NP, 2 * W), jnp.float32),
        mesh=_mesh,
        compiler_params=_sc_params,
        scratch_types=(
            pltpu.VMEM((NP,), jnp.float32),
            pltpu.VMEM((NP,), jnp.float32),
            pltpu.VMEM((PER_W, DEG), jnp.int32),
            pltpu.VMEM((PER_W, DEG), jnp.float32),
            pltpu.VMEM((512,), jnp.int32),
            pltpu.VMEM((512,), jnp.float32),
            pltpu.VMEM((256,), jnp.float32),
            pltpu.VMEM((LANES,), jnp.float32),
            pltpu.VMEM((2 * LANES,), jnp.int32),
            pltpu.VMEM((2 * DEG, W), jnp.float32),
            pltpu.VMEM((512, W), jnp.float32),
            pltpu.VMEM((4 * W,), jnp.float32),
            pltpu.SemaphoreType.DMA,
            pltpu.SemaphoreType.DMA,
            pltpu.SemaphoreType.DMA,
            pltpu.SemaphoreType.DMA,
            pltpu.SemaphoreType.DMA,
        ),
    )
    def layer(table, nbr2d, cand, w1_in, w2_in, d1_hbm, d2_hbm,
              out_hbm, d1v, d2v, nbrslab, w1slab, idx256, w2buf, coef2,
              coef1, idx16, rows16, rows256, outbuf, semA, semBC, semCand,
              semW2, semOut):
        wid = lax.axis_index("s") * NC + lax.axis_index("c")
        base = wid * PER_W
        pltpu.sync_copy(d1_hbm, d1v)
        pltpu.sync_copy(d2_hbm, d2v)
        pltpu.sync_copy(nbr2d.at[pl.ds(base, PER_W)], nbrslab)
        pltpu.sync_copy(w1_in.at[pl.ds(base, PER_W)], w1slab)

        iota = _iota()

        def issue_meta(node, s):
            pltpu.async_copy(cand.at[node], idx256.at[pl.ds(s * 256, 256)],
                             semCand)
            pltpu.async_copy(w2_in.at[node], w2buf.at[pl.ds(s * 256, 256)],
                             semW2)

        def wait_meta(s):
            pltpu.make_async_copy(cand.at[base],
                                  idx256.at[pl.ds(s * 256, 256)],
                                  semCand).wait()
            pltpu.make_async_copy(w2_in.at[base],
                                  w2buf.at[pl.ds(s * 256, 256)],
                                  semW2).wait()

        def issue_rows(node_l, s):
            v = plsc.load_gather(nbrslab, [_full(node_l), iota])
            idx16[pl.ds(s * LANES, LANES)] = v
            pltpu.async_copy(table.at[idx16.at[pl.ds(s * LANES, LANES)]],
                             rows16.at[pl.ds(s * DEG, DEG)], semA)
            pltpu.async_copy(
                table.at[idx256.at[pl.ds(s * 256, 128)]],
                rows256.at[pl.ds(s * 256, 128)], semBC)
            pltpu.async_copy(
                table.at[idx256.at[pl.ds(s * 256 + 128, 128)]],
                rows256.at[pl.ds(s * 256 + 128, 128)], semBC)

        def wait_rows(s):
            pltpu.make_async_copy(table.at[idx16.at[pl.ds(s * LANES, LANES)]],
                                  rows16.at[pl.ds(s * DEG, DEG)], semA).wait()
            for h in range(2):
                pltpu.make_async_copy(
                    table.at[idx256.at[pl.ds(s * 256 + h * 128, 128)]],
                    rows256.at[pl.ds(s * 256 + h * 128, 128)], semBC).wait()

        def out_slice(s):
            return outbuf.at[pl.ds(s * 2 * W, 2 * W)]

        issue_meta(base, 0)
        issue_meta(base + 1, 1)
        wait_meta(0)
        issue_rows(0, 0)

        def body(li, _):
            po = lax.rem(li, 2)
            pn = lax.rem(li + 1, 2)
            lip1 = jnp.minimum(li + 1, PER_W - 1)
            lip2 = jnp.minimum(li + 2, PER_W - 1)
            i = base + li
            p256 = po * 256

            wait_meta(pn)
            issue_rows(lip1, pn)

            v = plsc.load_gather(nbrslab, [_full(li), iota])
            w1v = plsc.load_gather(w1slab, [_full(li), iota])
            coef1[...] = w1v * plsc.load_gather(d1v, [v])

            def cg(g, _):
                cv = idx256[pl.ds(p256 + g * LANES, LANES)]
                coef2[pl.ds(g * LANES, LANES)] = (
                    w2buf[pl.ds(p256 + g * LANES, LANES)]
                    * plsc.load_gather(d2v, [cv]))
                return 0
            lax.fori_loop(0, 16, cg, 0)

            wait_rows(po)

            acc1 = [jnp.zeros((LANES,), jnp.float32) for _ in range(WL)]
            for t in range(DEG):
                sp = plsc.load_gather(coef1, [_full(t)])
                for l in range(WL):
                    acc1[l] = acc1[l] + sp * plsc.load_gather(
                        rows16, [_full(po * DEG + t), iota + l * LANES])

            def ag(g, accs):
                accs = list(accs)
                for t in range(16):
                    slot = g * 16 + t
                    sp = plsc.load_gather(coef2, [_full(slot)])
                    for l in range(WL):
                        accs[l] = accs[l] + sp * plsc.load_gather(
                            rows256, [_full(p256 + slot), iota + l * LANES])
                return tuple(accs)
            acc2 = lax.fori_loop(
                0, 16, ag,
                tuple(jnp.zeros((LANES,), jnp.float32) for _ in range(WL)))

            @pl.when(li >= 2)
            def _():
                pltpu.make_async_copy(out_slice(po), out_hbm.at[i],
                                      semOut).wait()

            d1i = plsc.load_gather(d1v, [_full(i)])
            d2i = plsc.load_gather(d2v, [_full(i)])
            for l in range(WL):
                outbuf[pl.ds(po * 2 * W + l * LANES, LANES)] = jnp.maximum(
                    acc1[l] * d1i, 0.0)
                outbuf[pl.ds(po * 2 * W + W + l * LANES, LANES)] = jnp.maximum(
                    acc2[l] * d2i, 0.0)
            pltpu.async_copy(out_slice(po), out_hbm.at[i], semOut)

            issue_meta(base + lip2, po)
            return 0

        lax.fori_loop(0, PER_W, body, 0)

        wait_meta(0)
        wait_rows(1)
        for s in range(2):
            pltpu.make_async_copy(out_slice(s), out_hbm.at[base], semOut).wait()

    return layer


_layer64 = _make_layer_kernel(HID)
_layer128 = _make_layer_kernel(2 * HID)


_BK = 1024


def _embed_body(x_ref, w_ref, o_ref):
    o_ref[...] = jnp.maximum(
        jnp.dot(x_ref[...], w_ref[...], preferred_element_type=jnp.float32),
        0.0)


def _classify_body(r0_ref, r1_ref, r2_ref, w_ref, o_ref):
    w = w_ref[...]
    lg = jnp.dot(r0_ref[...], w[0:HID],
                 preferred_element_type=jnp.float32)
    lg = lg + jnp.dot(r1_ref[...], w[HID:3 * HID],
                      preferred_element_type=jnp.float32)
    lg = lg + jnp.dot(r2_ref[...], w[3 * HID:7 * HID],
                      preferred_element_type=jnp.float32)
    m = jnp.max(lg, axis=1, keepdims=True)
    e = jnp.exp(lg - m)
    o_ref[...] = e / jnp.sum(e, axis=1, keepdims=True)


def kernel(x, edge_index, w_embed, w_classify):
    src = edge_index[1].astype(jnp.int32)
    nbr2d = jnp.zeros((NP, DEG), jnp.int32).at[:N].set(src.reshape(N, DEG))
    lut = jnp.where(jnp.arange(LUT) > 0,
                    jnp.arange(LUT, dtype=jnp.float32) ** -0.5,
                    0.0).astype(jnp.float32)
    x_pad = jnp.zeros((NP, F_IN), x.dtype).at[:N].set(x)

    r0 = pl.pallas_call(
        _embed_body,
        grid=(NP // _BK,),
        in_specs=[
            pl.BlockSpec((_BK, F_IN), lambda g: (g, 0)),
            pl.BlockSpec((F_IN, HID), lambda g: (0, 0)),
        ],
        out_specs=pl.BlockSpec((_BK, HID), lambda g: (g, 0)),
        out_shape=jax.ShapeDtypeStruct((NP, HID), jnp.float32),
    )(x_pad, w_embed)

    cand, w1, w2, d1, d2 = _structure_kernel(nbr2d, lut)

    r1 = _layer64(r0, nbr2d, cand, w1, w2, d1, d2)
    r2 = _layer128(r1, nbr2d, cand, w1, w2, d1, d2)

    out = pl.pallas_call(
        _classify_body,
        grid=(NP // _BK,),
        in_specs=[
            pl.BlockSpec((_BK, HID), lambda g: (g, 0)),
            pl.BlockSpec((_BK, 2 * HID), lambda g: (g, 0)),
            pl.BlockSpec((_BK, 4 * HID), lambda g: (g, 0)),
            pl.BlockSpec((7 * HID, N_CLS), lambda g: (0, 0)),
        ],
        out_specs=pl.BlockSpec((_BK, N_CLS), lambda g: (g, 0)),
        out_shape=jax.ShapeDtypeStruct((NP, N_CLS), jnp.float32),
    )(r0, r1, r2, w_classify)

    return out[:N]


# speedup vs baseline: 62.9122x; 1.3400x over previous
import functools

import jax
import jax.numpy as jnp
from jax import lax
from jax.experimental import pallas as pl
from jax.experimental.pallas import tpu as pltpu
from jax.experimental.pallas import tpu_sc as plsc

N = 10000
DEG = 16
F_IN = 128
HID = 64
N_CLS = 10

NC = 2
NS = 16
NW = NC * NS
NP = 10240
PER_W = NP // NW
LANES = 16
LUT = 320

_mesh = plsc.VectorSubcoreMesh(
    core_axis_name="c", subcore_axis_name="s", num_cores=NC, num_subcores=NS)
_sc_params = pltpu.CompilerParams(
    needs_layout_passes=False, use_tc_tiling_on_sc=False)


def _iota():
    return lax.iota(jnp.int32, LANES)


def _full(v):
    return jnp.full((LANES,), v, jnp.int32)


@functools.partial(
    pl.kernel,
    out_type=(
        jax.ShapeDtypeStruct((NP, 256), jnp.int32),
        jax.ShapeDtypeStruct((NP, DEG), jnp.float32),
        jax.ShapeDtypeStruct((NP, 256), jnp.float32),
        jax.ShapeDtypeStruct((NP,), jnp.float32),
        jax.ShapeDtypeStruct((NP,), jnp.float32),
    ),
    mesh=_mesh,
    compiler_params=_sc_params,
    scratch_types=(
        pltpu.VMEM((NP,), jnp.int32),
        pltpu.VMEM((NP,), jnp.int32),
        pltpu.VMEM((PER_W, DEG), jnp.int32),
        pltpu.VMEM((LANES, LANES), jnp.int32),
        pltpu.VMEM((256,), jnp.int32),
        pltpu.VMEM((256,), jnp.float32),
        pltpu.VMEM((PER_W, DEG), jnp.float32),
        pltpu.VMEM((PER_W,), jnp.float32),
        pltpu.VMEM((PER_W,), jnp.float32),
        pltpu.VMEM((LUT,), jnp.float32),
        pltpu.VMEM((LANES,), jnp.int32),
        pltpu.SemaphoreType.DMA,
    ),
)
def _structure_kernel(nbr2d, lut_hbm, cand_out, w1_out, w2_out,
                      d1_out, d2_out, cnt, slot, nbrslab, cand2d, candbuf,
                      w2buf, w1slab, d1slab, d2slab, lutv, idx16, sem):
    wid = lax.axis_index("s") * NC + lax.axis_index("c")
    base = wid * PER_W
    pltpu.sync_copy(lut_hbm, lutv)
    pltpu.sync_copy(nbr2d.at[pl.ds(base, PER_W)], nbrslab)

    iota = _iota()
    lane0 = iota == 0
    zeros_i = jnp.zeros((LANES,), jnp.int32)
    ones_i = jnp.ones((LANES,), jnp.int32)

    def _zb(j, _):
        plsc.store_scatter(cnt, [iota + j * LANES], zeros_i)
        return 0
    lax.fori_loop(0, NP // LANES, _zb, 0)

    def body(li, _):
        i = base + li
        i_spl = _full(i)
        v = plsc.load_gather(nbrslab, [_full(li), iota])
        idx16[...] = v
        cp = pltpu.async_copy(nbr2d.at[idx16], cand2d, sem)

        plsc.addupdate_scatter(cnt, [v], ones_i)
        g = plsc.load_gather(cnt, [v])
        plsc.store_scatter(slot, [v], iota)
        back = plsc.load_gather(slot, [v])
        chosen = back == iota
        g_adj = g - jnp.where(v == i_spl, 1, 0)
        valid1 = chosen & (g_adj > 0)
        w1v = jnp.where(valid1, 1.0, 0.0)
        plsc.store_scatter(w1slab, [_full(li), iota], w1v)
        deg1 = plsc.all_reduce_population_count(valid1)
        plsc.store_scatter(cnt, [v], zeros_i)

        cp.wait()

        for s in range(16):
            cv = plsc.load_gather(cand2d, [_full(s), iota])
            candbuf[pl.ds(s * LANES, LANES)] = cv
            plsc.addupdate_scatter(cnt, [cv], ones_i)
        plsc.addupdate_scatter(cnt, [v], -ones_i)
        plsc.addupdate_scatter(cnt, [i_spl], -ones_i, mask=lane0)
        for s in range(16):
            cv = candbuf[pl.ds(s * LANES, LANES)]
            plsc.store_scatter(slot, [cv], iota + 16 * s)
        deg2 = jnp.zeros((LANES,), jnp.int32)
        for s in range(16):
            cv = candbuf[pl.ds(s * LANES, LANES)]
            g2 = plsc.load_gather(cnt, [cv])
            b2 = plsc.load_gather(slot, [cv])
            m = (b2 == iota + 16 * s) & (g2 > 0)
            w2buf[pl.ds(s * LANES, LANES)] = jnp.where(m, 1.0, 0.0)
            deg2 = deg2 + plsc.all_reduce_population_count(m)
        for s in range(16):
            cv = candbuf[pl.ds(s * LANES, LANES)]
            plsc.store_scatter(cnt, [cv], zeros_i)
        plsc.store_scatter(cnt, [v], zeros_i)
        plsc.store_scatter(cnt, [i_spl], zeros_i, mask=lane0)

        d1s = plsc.load_gather(lutv, [deg1])
        d2s = plsc.load_gather(lutv, [deg2])
        plsc.store_scatter(d1slab, [_full(li)], d1s, mask=lane0)
        plsc.store_scatter(d2slab, [_full(li)], d2s, mask=lane0)

        pltpu.sync_copy(candbuf, cand_out.at[i])
        pltpu.sync_copy(w2buf, w2_out.at[i])
        return 0

    lax.fori_loop(0, PER_W, body, 0)
    pltpu.sync_copy(w1slab, w1_out.at[pl.ds(base, PER_W)])
    pltpu.sync_copy(d1slab, d1_out.at[pl.ds(base, PER_W)])
    pltpu.sync_copy(d2slab, d2_out.at[pl.ds(base, PER_W)])


def _make_layer_kernel(W):
    WL = W // LANES

    @functools.partial(
        pl.kernel,
        out_type=jax.ShapeDtypeStruct((NP, 2 * W), jnp.float32),
        mesh=_mesh,
        compiler_params=_sc_params,
        scratch_types=(
            pltpu.VMEM((NP,), jnp.float32),
            pltpu.VMEM((NP,), jnp.float32),
            pltpu.VMEM((PER_W, DEG), jnp.int32),
            pltpu.VMEM((PER_W, DEG), jnp.float32),
            pltpu.VMEM((512,), jnp.int32),
            pltpu.VMEM((512,), jnp.float32),
            pltpu.VMEM((256,), jnp.float32),
            pltpu.VMEM((LANES,), jnp.float32),
            pltpu.VMEM((2 * LANES,), jnp.int32),
            pltpu.VMEM((2 * DEG, W), jnp.float32),
            pltpu.VMEM((512, W), jnp.float32),
            pltpu.VMEM((4 * W,), jnp.float32),
            pltpu.SemaphoreType.DMA,
            pltpu.SemaphoreType.DMA,
            pltpu.SemaphoreType.DMA,
            pltpu.SemaphoreType.DMA,
            pltpu.SemaphoreType.DMA,
        ),
    )
    def layer(table, nbr2d, cand, w1_in, w2_in, d1_hbm, d2_hbm,
              out_hbm, d1v, d2v, nbrslab, w1slab, idx256, w2buf, coef2,
              coef1, idx16, rows16, rows256, outbuf, semA, semBC, semCand,
              semW2, semOut):
        wid = lax.axis_index("s") * NC + lax.axis_index("c")
        base = wid * PER_W
        pltpu.sync_copy(d1_hbm, d1v)
        pltpu.sync_copy(d2_hbm, d2v)
        pltpu.sync_copy(nbr2d.at[pl.ds(base, PER_W)], nbrslab)
        pltpu.sync_copy(w1_in.at[pl.ds(base, PER_W)], w1slab)

        iota = _iota()

        def issue_meta(node, s):
            pltpu.async_copy(cand.at[node], idx256.at[pl.ds(s * 256, 256)],
                             semCand)
            pltpu.async_copy(w2_in.at[node], w2buf.at[pl.ds(s * 256, 256)],
                             semW2)

        def wait_meta(s):
            pltpu.make_async_copy(cand.at[base],
                                  idx256.at[pl.ds(s * 256, 256)],
                                  semCand).wait()
            pltpu.make_async_copy(w2_in.at[base],
                                  w2buf.at[pl.ds(s * 256, 256)],
                                  semW2).wait()

        def issue_rows(node_l, s):
            v = plsc.load_gather(nbrslab, [_full(node_l), iota])
            idx16[pl.ds(s * LANES, LANES)] = v
            pltpu.async_copy(table.at[idx16.at[pl.ds(s * LANES, LANES)]],
                             rows16.at[pl.ds(s * DEG, DEG)], semA)
            pltpu.async_copy(
                table.at[idx256.at[pl.ds(s * 256, 128)]],
                rows256.at[pl.ds(s * 256, 128)], semBC)
            pltpu.async_copy(
                table.at[idx256.at[pl.ds(s * 256 + 128, 128)]],
                rows256.at[pl.ds(s * 256 + 128, 128)], semBC)

        def wait_rows(s):
            pltpu.make_async_copy(table.at[idx16.at[pl.ds(s * LANES, LANES)]],
                                  rows16.at[pl.ds(s * DEG, DEG)], semA).wait()
            for h in range(2):
                pltpu.make_async_copy(
                    table.at[idx256.at[pl.ds(s * 256 + h * 128, 128)]],
                    rows256.at[pl.ds(s * 256 + h * 128, 128)], semBC).wait()

        def out_slice(s):
            return outbuf.at[pl.ds(s * 2 * W, 2 * W)]

        issue_meta(base, 0)
        issue_meta(base + 1, 1)
        wait_meta(0)
        issue_rows(0, 0)

        def body(li, _):
            po = lax.rem(li, 2)
            pn = lax.rem(li + 1, 2)
            lip1 = jnp.minimum(li + 1, PER_W - 1)
            lip2 = jnp.minimum(li + 2, PER_W - 1)
            i = base + li
            p256 = po * 256

            wait_meta(pn)
            issue_rows(lip1, pn)

            v = plsc.load_gather(nbrslab, [_full(li), iota])
            w1v = plsc.load_gather(w1slab, [_full(li), iota])
            coef1[...] = w1v * plsc.load_gather(d1v, [v])

            def cg(g, _):
                cv = idx256[pl.ds(p256 + g * LANES, LANES)]
                coef2[pl.ds(g * LANES, LANES)] = (
                    w2buf[pl.ds(p256 + g * LANES, LANES)]
                    * plsc.load_gather(d2v, [cv]))
                return 0
            lax.fori_loop(0, 16, cg, 0)

            wait_rows(po)

            acc1 = [jnp.zeros((LANES,), jnp.float32) for _ in range(WL)]
            for t in range(DEG):
                sp = plsc.load_gather(coef1, [_full(t)])
                row = po * DEG + t
                for l in range(WL):
                    acc1[l] = acc1[l] + sp * rows16[row, pl.ds(l * LANES, LANES)]

            def ag(g, accs):
                accs = list(accs)
                for t in range(16):
                    slot = g * 16 + t
                    sp = plsc.load_gather(coef2, [_full(slot)])
                    row = p256 + slot
                    for l in range(WL):
                        accs[l] = accs[l] + sp * rows256[row,
                                                         pl.ds(l * LANES, LANES)]
                return tuple(accs)
            acc2 = lax.fori_loop(
                0, 16, ag,
                tuple(jnp.zeros((LANES,), jnp.float32) for _ in range(WL)))

            @pl.when(li >= 2)
            def _():
                pltpu.make_async_copy(out_slice(po), out_hbm.at[i],
                                      semOut).wait()

            d1i = plsc.load_gather(d1v, [_full(i)])
            d2i = plsc.load_gather(d2v, [_full(i)])
            for l in range(WL):
                outbuf[pl.ds(po * 2 * W + l * LANES, LANES)] = jnp.maximum(
                    acc1[l] * d1i, 0.0)
                outbuf[pl.ds(po * 2 * W + W + l * LANES, LANES)] = jnp.maximum(
                    acc2[l] * d2i, 0.0)
            pltpu.async_copy(out_slice(po), out_hbm.at[i], semOut)

            issue_meta(base + lip2, po)
            return 0

        lax.fori_loop(0, PER_W, body, 0)

        wait_meta(0)
        wait_rows(1)
        for s in range(2):
            pltpu.make_async_copy(out_slice(s), out_hbm.at[base], semOut).wait()

    return layer


_layer64 = _make_layer_kernel(HID)
_layer128 = _make_layer_kernel(2 * HID)


_BK = 1024


def _embed_body(x_ref, w_ref, o_ref):
    o_ref[...] = jnp.maximum(
        jnp.dot(x_ref[...], w_ref[...], preferred_element_type=jnp.float32),
        0.0)


def _classify_body(r0_ref, r1_ref, r2_ref, w_ref, o_ref):
    w = w_ref[...]
    lg = jnp.dot(r0_ref[...], w[0:HID],
                 preferred_element_type=jnp.float32)
    lg = lg + jnp.dot(r1_ref[...], w[HID:3 * HID],
                      preferred_element_type=jnp.float32)
    lg = lg + jnp.dot(r2_ref[...], w[3 * HID:7 * HID],
                      preferred_element_type=jnp.float32)
    m = jnp.max(lg, axis=1, keepdims=True)
    e = jnp.exp(lg - m)
    o_ref[...] = e / jnp.sum(e, axis=1, keepdims=True)


def kernel(x, edge_index, w_embed, w_classify):
    src = edge_index[1].astype(jnp.int32)
    nbr2d = jnp.zeros((NP, DEG), jnp.int32).at[:N].set(src.reshape(N, DEG))
    lut = jnp.where(jnp.arange(LUT) > 0,
                    jnp.arange(LUT, dtype=jnp.float32) ** -0.5,
                    0.0).astype(jnp.float32)
    x_pad = jnp.zeros((NP, F_IN), x.dtype).at[:N].set(x)

    r0 = pl.pallas_call(
        _embed_body,
        grid=(NP // _BK,),
        in_specs=[
            pl.BlockSpec((_BK, F_IN), lambda g: (g, 0)),
            pl.BlockSpec((F_IN, HID), lambda g: (0, 0)),
        ],
        out_specs=pl.BlockSpec((_BK, HID), lambda g: (g, 0)),
        out_shape=jax.ShapeDtypeStruct((NP, HID), jnp.float32),
    )(x_pad, w_embed)

    cand, w1, w2, d1, d2 = _structure_kernel(nbr2d, lut)

    r1 = _layer64(r0, nbr2d, cand, w1, w2, d1, d2)
    r2 = _layer128(r1, nbr2d, cand, w1, w2, d1, d2)

    out = pl.pallas_call(
        _classify_body,
        grid=(NP // _BK,),
        in_specs=[
            pl.BlockSpec((_BK, HID), lambda g: (g, 0)),
            pl.BlockSpec((_BK, 2 * HID), lambda g: (g, 0)),
            pl.BlockSpec((_BK, 4 * HID), lambda g: (g, 0)),
            pl.BlockSpec((7 * HID, N_CLS), lambda g: (0, 0)),
        ],
        out_specs=pl.BlockSpec((_BK, N_CLS), lambda g: (g, 0)),
        out_shape=jax.ShapeDtypeStruct((NP, N_CLS), jnp.float32),
    )(r0, r1, r2, w_classify)

    return out[:N]
